# baseline (speedup 1.0000x reference)
import jax
import jax.numpy as jnp
from jax.experimental import pallas as pl
from jax.experimental.pallas import tpu as pltpu

_B = 16
_S = 64
_R = _S - 1
_D = 1024
_DSPLIT = 4
_DW = _D // _DSPLIT


def kernel(arr):
    B, S2, D = arr.shape

    def body(in_ref, out_ref):
        for i in range(_R):
            out_ref[0, i * _S : (i + 1) * _S, :] = in_ref[
                0, i * (_S + 1) + 1 : i * (_S + 1) + 1 + _S, :
            ]

    out = pl.pallas_call(
        body,
        grid=(B, _DSPLIT),
        in_specs=[
            pl.BlockSpec((1, S2, _DW), lambda b, d: (b, 0, d)),
        ],
        out_specs=pl.BlockSpec((1, _R * _S, _DW), lambda b, d: (b, 0, d)),
        out_shape=jax.ShapeDtypeStruct((B, _R * _S, D), jnp.float32),
    )(arr)
    return out


# speedup vs baseline: 1.0167x; 1.0167x over previous
import jax
import jax.numpy as jnp
from jax.experimental import pallas as pl

_S = 64
_R = _S - 1
_DSPLIT = 2


def kernel(arr):
    B, S2, D = arr.shape
    dw = D // _DSPLIT

    def body(in_ref, out_ref):
        for i in range(_R):
            out_ref[0, i * _S : (i + 1) * _S, :] = in_ref[
                0, i * (_S + 1) + 1 : i * (_S + 1) + 1 + _S, :
            ]

    return pl.pallas_call(
        body,
        grid=(B, _DSPLIT),
        in_specs=[pl.BlockSpec((1, S2, dw), lambda b, d: (b, 0, d))],
        out_specs=pl.BlockSpec((1, _R * _S, dw), lambda b, d: (b, 0, d)),
        out_shape=jax.ShapeDtypeStruct((B, _R * _S, D), jnp.float32),
    )(arr)
